# Initial kernel scaffold; baseline (speedup 1.0000x reference)
import jax
import jax.numpy as jnp
from jax.experimental import pallas as pl


def kernel(gene_data, image_data, gene_edge_index, spatial_edge_index, fc1_W, fc1_b, fc2_W, fc2_b, g11_W, g11_b, g12_W, g12_b, g21_W, g21_b, g22_W, g22_b, w1, w2):
    raise NotImplementedError("write your pallas kernel here")


# speedup vs baseline: 30.9745x; 30.9745x over previous
import functools

import jax
import jax.numpy as jnp
from jax import lax
from jax.experimental import pallas as pl
from jax.experimental.pallas import tpu as pltpu
from jax.experimental.pallas import tpu_sc as plsc

N = 50000
E = 800000
H = 64
H2 = 32
OUT = 64

NTILE = 16
NCORE = 2
CHUNK = 128
GK = 7
NG = 56
EPT = NG * GK * CHUNK
E_PAD = EPT * NTILE
CROWS = E_PAD // CHUNK
CPT = CROWS // NTILE
NP = 50048
ROW_A = 3128
ROW_B = N - 15 * ROW_A
ACC_R = NP // NTILE

_MESH = plsc.VectorSubcoreMesh(
    core_axis_name="c", subcore_axis_name="s", num_cores=NCORE,
    num_subcores=NTILE)

_SC_PARAMS = pltpu.CompilerParams(use_tc_tiling_on_sc=False)


def _edge_groups(idx_hbm, tile):
    base = tile * CPT

    def at(g):
        return idx_hbm.at[pl.ds(pl.multiple_of(base + g * GK, 8), GK)]

    return at


def _sliced_copy(get_src, get_dst, tile):
    @pl.when(tile < NTILE - 1)
    def _():
        off = pl.multiple_of(tile * ROW_A, 8)
        pltpu.sync_copy(get_src(off, ROW_A), get_dst(off, ROW_A))

    @pl.when(tile == NTILE - 1)
    def _():
        off = (NTILE - 1) * ROW_A
        pltpu.sync_copy(get_src(off, ROW_B), get_dst(off, ROW_B))


@functools.partial(
    pl.kernel,
    out_type=(
        jax.ShapeDtypeStruct((N, 8), jnp.float32),
        jax.ShapeDtypeStruct((N, 8), jnp.float32),
    ),
    mesh=_MESH,
    scratch_types=(
        pltpu.VMEM((GK, CHUNK), jnp.int32),
        pltpu.VMEM((CHUNK, 8), jnp.float32),
        pltpu.VMEM_SHARED((NP, 8), jnp.float32),
        pltpu.SemaphoreType.DMA,
    ),
    compiler_params=_SC_PARAMS,
)
def _deg_kernel(dst_a, dst_b, ones_init, ones_tbl, deg_a, deg_b,
                dst_v, ones_v, acc, sem):
    cid = lax.axis_index("c")
    tile = lax.axis_index("s")
    pltpu.sync_copy(ones_tbl, ones_v)

    def run(dst_hbm, out_hbm):
        ioff = pl.multiple_of(tile * ACC_R, 8)
        pltpu.sync_copy(ones_init.at[pl.ds(ioff, ACC_R)],
                        acc.at[pl.ds(ioff, ACC_R)])
        plsc.subcore_barrier()
        src_at = _edge_groups(dst_hbm, tile)

        def group(g, carry):
            pltpu.sync_copy(src_at(g), dst_v)
            descs = [
                pltpu.async_copy(ones_v, acc.at[dst_v.at[k]], sem, add=True)
                for k in range(GK)
            ]
            for d in descs:
                d.wait()
            return carry

        lax.fori_loop(0, NG, group, 0)
        plsc.subcore_barrier()
        _sliced_copy(lambda o, n: acc.at[pl.ds(o, n)],
                     lambda o, n: out_hbm.at[pl.ds(o, n)], tile)

    pl.when(cid == 0)(lambda: run(dst_a, deg_a))
    pl.when(cid == 1)(lambda: run(dst_b, deg_b))


@functools.partial(
    pl.kernel,
    out_type=(
        jax.ShapeDtypeStruct((N, H2), jnp.float32),
        jax.ShapeDtypeStruct((N, H2), jnp.float32),
    ),
    mesh=_MESH,
    scratch_types=(
        pltpu.VMEM((GK, CHUNK), jnp.int32),
        pltpu.VMEM((GK, CHUNK), jnp.int32),
        pltpu.VMEM((GK, CHUNK, H2), jnp.float32),
        pltpu.VMEM_SHARED((NP, H2), jnp.float32),
        pltpu.SemaphoreType.DMA,
        pltpu.SemaphoreType.DMA,
    ),
    compiler_params=_SC_PARAMS,
)
def _prop_kernel(z_a, z_b, src_a, dst_a, src_b, dst_b, u_a, u_b,
                 src_v, dst_v, rows, acc, gsem, ssem):
    cid = lax.axis_index("c")
    tile = lax.axis_index("s")

    def run(z_hbm, src_hbm, dst_hbm, out_hbm):
        _sliced_copy(lambda o, n: z_hbm.at[pl.ds(o, n)],
                     lambda o, n: acc.at[pl.ds(o, n)], tile)
        plsc.subcore_barrier()
        src_at = _edge_groups(src_hbm, tile)
        dst_at = _edge_groups(dst_hbm, tile)

        def group(g, carry):
            pltpu.sync_copy(src_at(g), src_v)
            pltpu.sync_copy(dst_at(g), dst_v)
            gds = [
                pltpu.async_copy(z_hbm.at[src_v.at[k]], rows.at[k], gsem)
                for k in range(GK)
            ]
            for d in gds:
                d.wait()
            sds = [
                pltpu.async_copy(rows.at[k], acc.at[dst_v.at[k]], ssem,
                                 add=True)
                for k in range(GK)
            ]
            for d in sds:
                d.wait()
            return carry

        lax.fori_loop(0, NG, group, 0)
        plsc.subcore_barrier()
        _sliced_copy(lambda o, n: acc.at[pl.ds(o, n)],
                     lambda o, n: out_hbm.at[pl.ds(o, n)], tile)

    pl.when(cid == 0)(lambda: run(z_a, src_a, dst_a, u_a))
    pl.when(cid == 1)(lambda: run(z_b, src_b, dst_b, u_b))


R = 1000
GRID = N // R


def _enc_body(gene, image, fc1w, fc1b, fc2w, fc2b, g11w, g12w, degs, degg,
              z1, z2):
    dinv_s = lax.rsqrt(degs[...])
    dinv_g = lax.rsqrt(degg[...])
    h1 = jnp.maximum(
        jnp.dot(gene[...], fc1w[...], preferred_element_type=jnp.float32)
        + fc1b[...], 0.0)
    h2 = jnp.maximum(
        jnp.dot(image[...], fc2w[...], preferred_element_type=jnp.float32)
        + fc2b[...], 0.0)
    z1[...] = jnp.dot(h1, g11w[...], preferred_element_type=jnp.float32) * dinv_s
    z2[...] = jnp.dot(h2, g12w[...], preferred_element_type=jnp.float32) * dinv_g


def _mid_body(u1, u2, degs, degg, g11b, g12b, g21w, g22w, w1,
              y3a, y3b, y4a, y4b):
    dinv_s = lax.rsqrt(degs[...])
    dinv_g = lax.rsqrt(degg[...])
    x1 = jnp.maximum(u1[...] * dinv_s + g11b[...], 0.0)
    x2 = jnp.maximum(u2[...] * dinv_g + g12b[...], 0.0)
    a = w1[0, 0]
    x = a * x1 + (1.0 - a) * x2
    y3 = jnp.dot(x, g21w[...], preferred_element_type=jnp.float32) * dinv_s
    y4 = jnp.dot(x, g22w[...], preferred_element_type=jnp.float32) * dinv_g
    y3a[...] = y3[:, :H2]
    y3b[...] = y3[:, H2:]
    y4a[...] = y4[:, :H2]
    y4b[...] = y4[:, H2:]


def _out_body(u3a, u3b, u4a, u4b, degs, degg, g21b, g22b, w2, out):
    dinv_s = lax.rsqrt(degs[...])
    dinv_g = lax.rsqrt(degg[...])
    u3 = jnp.concatenate([u3a[...], u3b[...]], axis=1)
    u4 = jnp.concatenate([u4a[...], u4b[...]], axis=1)
    a = w2[0, 0]
    out[...] = (a * (u3 * dinv_s + g21b[...])
                + (1.0 - a) * (u4 * dinv_g + g22b[...]))


def _row_spec(cols):
    return pl.BlockSpec((R, cols), lambda i: (i, 0))


def _full_spec(r, c):
    return pl.BlockSpec((r, c), lambda i: (0, 0))


_enc_call = pl.pallas_call(
    _enc_body,
    grid=(GRID,),
    in_specs=[
        _row_spec(512), _row_spec(128),
        _full_spec(512, H), _full_spec(1, H),
        _full_spec(128, H), _full_spec(1, H),
        _full_spec(H, H2), _full_spec(H, H2),
        _row_spec(1), _row_spec(1),
    ],
    out_specs=[_row_spec(H2), _row_spec(H2)],
    out_shape=[
        jax.ShapeDtypeStruct((N, H2), jnp.float32),
        jax.ShapeDtypeStruct((N, H2), jnp.float32),
    ],
)

_mid_call = pl.pallas_call(
    _mid_body,
    grid=(GRID,),
    in_specs=[
        _row_spec(H2), _row_spec(H2), _row_spec(1), _row_spec(1),
        _full_spec(1, H2), _full_spec(1, H2),
        _full_spec(H2, H), _full_spec(H2, H),
        _full_spec(1, 1),
    ],
    out_specs=[_row_spec(H2)] * 4,
    out_shape=[jax.ShapeDtypeStruct((N, H2), jnp.float32)] * 4,
)

_out_call = pl.pallas_call(
    _out_body,
    grid=(GRID,),
    in_specs=[
        _row_spec(H2), _row_spec(H2), _row_spec(H2), _row_spec(H2),
        _row_spec(1), _row_spec(1),
        _full_spec(1, OUT), _full_spec(1, OUT),
        _full_spec(1, 1),
    ],
    out_specs=_row_spec(OUT),
    out_shape=jax.ShapeDtypeStruct((N, OUT), jnp.float32),
)


def _pad_edges(edge_index):
    pad_src = jnp.zeros((E_PAD - E,), jnp.int32)
    pad_dst = jnp.full((E_PAD - E,), N, jnp.int32)
    src = jnp.concatenate([edge_index[0], pad_src]).reshape(CROWS, CHUNK)
    dst = jnp.concatenate([edge_index[1], pad_dst]).reshape(CROWS, CHUNK)
    return src, dst


def kernel(gene_data, image_data, gene_edge_index, spatial_edge_index,
           fc1_W, fc1_b, fc2_W, fc2_b,
           g11_W, g11_b, g12_W, g12_b,
           g21_W, g21_b, g22_W, g22_b, w1, w2):
    src_s, dst_s = _pad_edges(spatial_edge_index)
    src_g, dst_g = _pad_edges(gene_edge_index)
    ones_init = jnp.ones((NP, 8), jnp.float32)
    ones_tbl = jnp.ones((CHUNK, 8), jnp.float32)

    deg_s8, deg_g8 = _deg_kernel(dst_s, dst_g, ones_init, ones_tbl)
    deg_s = deg_s8[:, :1]
    deg_g = deg_g8[:, :1]

    z1, z2 = _enc_call(gene_data, image_data,
                       fc1_W, fc1_b.reshape(1, H), fc2_W, fc2_b.reshape(1, H),
                       g11_W, g12_W, deg_s, deg_g)
    u1, u2 = _prop_kernel(z1, z2, src_s, dst_s, src_g, dst_g)

    y3a, y3b, y4a, y4b = _mid_call(
        u1, u2, deg_s, deg_g,
        g11_b.reshape(1, H2), g12_b.reshape(1, H2),
        g21_W, g22_W, jnp.reshape(w1, (1, 1)))

    u3a, u4a = _prop_kernel(y3a, y4a, src_s, dst_s, src_g, dst_g)
    u3b, u4b = _prop_kernel(y3b, y4b, src_s, dst_s, src_g, dst_g)

    return _out_call(u3a, u3b, u4a, u4b, deg_s, deg_g,
                     g21_b.reshape(1, OUT), g22_b.reshape(1, OUT),
                     jnp.reshape(w2, (1, 1)))


# speedup vs baseline: 33.0056x; 1.0656x over previous
import functools

import jax
import jax.numpy as jnp
from jax import lax
from jax.experimental import pallas as pl
from jax.experimental.pallas import tpu as pltpu
from jax.experimental.pallas import tpu_sc as plsc

N = 50000
E = 800000
H = 64
H2 = 32
OUT = 64

NTILE = 16
NCORE = 2
CHUNK = 128
GK = 7
NG = 56
EPT = NG * GK * CHUNK
E_PAD = EPT * NTILE
CROWS = E_PAD // CHUNK
CPT = CROWS // NTILE
NP = 50048
ROW_A = 3128
ROW_B = N - 15 * ROW_A
ACC_R = NP // NTILE

_MESH = plsc.VectorSubcoreMesh(
    core_axis_name="c", subcore_axis_name="s", num_cores=NCORE,
    num_subcores=NTILE)

_SC_PARAMS = pltpu.CompilerParams(use_tc_tiling_on_sc=False)


def _edge_groups(idx_hbm, tile):
    base = tile * CPT

    def at(g):
        return idx_hbm.at[pl.ds(pl.multiple_of(base + g * GK, 8), GK)]

    return at


def _sliced_copy(get_src, get_dst, tile):
    @pl.when(tile < NTILE - 1)
    def _():
        off = pl.multiple_of(tile * ROW_A, 8)
        pltpu.sync_copy(get_src(off, ROW_A), get_dst(off, ROW_A))

    @pl.when(tile == NTILE - 1)
    def _():
        off = (NTILE - 1) * ROW_A
        pltpu.sync_copy(get_src(off, ROW_B), get_dst(off, ROW_B))


@functools.partial(
    pl.kernel,
    out_type=(
        jax.ShapeDtypeStruct((N, 8), jnp.float32),
        jax.ShapeDtypeStruct((N, 8), jnp.float32),
    ),
    mesh=_MESH,
    scratch_types=(
        pltpu.VMEM((GK, CHUNK), jnp.int32),
        pltpu.VMEM((CHUNK, 8), jnp.float32),
        pltpu.VMEM_SHARED((NP, 8), jnp.float32),
        pltpu.SemaphoreType.DMA,
    ),
    compiler_params=_SC_PARAMS,
)
def _deg_kernel(dst_a, dst_b, ones_init, ones_tbl, deg_a, deg_b,
                dst_v, ones_v, acc, sem):
    cid = lax.axis_index("c")
    tile = lax.axis_index("s")
    pltpu.sync_copy(ones_tbl, ones_v)

    def run(dst_hbm, out_hbm):
        ioff = pl.multiple_of(tile * ACC_R, 8)
        pltpu.sync_copy(ones_init.at[pl.ds(ioff, ACC_R)],
                        acc.at[pl.ds(ioff, ACC_R)])
        plsc.subcore_barrier()
        src_at = _edge_groups(dst_hbm, tile)

        def group(g, carry):
            pltpu.sync_copy(src_at(g), dst_v)
            descs = [
                pltpu.async_copy(ones_v, acc.at[dst_v.at[k]], sem, add=True)
                for k in range(GK)
            ]
            for d in descs:
                d.wait()
            return carry

        lax.fori_loop(0, NG, group, 0)
        plsc.subcore_barrier()
        _sliced_copy(lambda o, n: acc.at[pl.ds(o, n)],
                     lambda o, n: out_hbm.at[pl.ds(o, n)], tile)

    pl.when(cid == 0)(lambda: run(dst_a, deg_a))
    pl.when(cid == 1)(lambda: run(dst_b, deg_b))


def _prop_core(z_hbm, src_hbm, dst_hbm, out_hbm,
               src_v, dst_v, rows, acc, gsem, ssem, tile):
    _sliced_copy(lambda o, n: z_hbm.at[pl.ds(o, n)],
                 lambda o, n: acc.at[pl.ds(o, n)], tile)
    plsc.subcore_barrier()
    src_at = _edge_groups(src_hbm, tile)
    dst_at = _edge_groups(dst_hbm, tile)

    def group(g, carry):
        pltpu.sync_copy(src_at(g), src_v)
        pltpu.sync_copy(dst_at(g), dst_v)
        gds = [
            pltpu.async_copy(z_hbm.at[src_v.at[k]], rows.at[k], gsem)
            for k in range(GK)
        ]
        sds = []
        for k in range(GK):
            gds[k].wait()
            sds.append(
                pltpu.async_copy(rows.at[k], acc.at[dst_v.at[k]], ssem,
                                 add=True))
        for d in sds:
            d.wait()
        return carry

    lax.fori_loop(0, NG, group, 0)
    plsc.subcore_barrier()
    _sliced_copy(lambda o, n: acc.at[pl.ds(o, n)],
                 lambda o, n: out_hbm.at[pl.ds(o, n)], tile)


_PROP_SCRATCH = (
    pltpu.VMEM((GK, CHUNK), jnp.int32),
    pltpu.VMEM((GK, CHUNK), jnp.int32),
    pltpu.VMEM((GK, CHUNK, H2), jnp.float32),
    pltpu.VMEM_SHARED((NP, H2), jnp.float32),
    pltpu.SemaphoreType.DMA,
    pltpu.SemaphoreType.DMA,
)


@functools.partial(
    pl.kernel,
    out_type=(
        jax.ShapeDtypeStruct((N, H2), jnp.float32),
        jax.ShapeDtypeStruct((N, H2), jnp.float32),
    ),
    mesh=_MESH,
    scratch_types=_PROP_SCRATCH,
    compiler_params=_SC_PARAMS,
)
def _prop_kernel(z_a, z_b, src_a, dst_a, src_b, dst_b, u_a, u_b,
                 src_v, dst_v, rows, acc, gsem, ssem):
    cid = lax.axis_index("c")
    tile = lax.axis_index("s")
    scr = (src_v, dst_v, rows, acc, gsem, ssem)
    pl.when(cid == 0)(lambda: _prop_core(z_a, src_a, dst_a, u_a, *scr, tile))
    pl.when(cid == 1)(lambda: _prop_core(z_b, src_b, dst_b, u_b, *scr, tile))


@functools.partial(
    pl.kernel,
    out_type=(
        jax.ShapeDtypeStruct((N, H2), jnp.float32),
        jax.ShapeDtypeStruct((N, H2), jnp.float32),
        jax.ShapeDtypeStruct((N, H2), jnp.float32),
        jax.ShapeDtypeStruct((N, H2), jnp.float32),
    ),
    mesh=_MESH,
    scratch_types=_PROP_SCRATCH,
    compiler_params=_SC_PARAMS,
)
def _prop2_kernel(z_a0, z_a1, z_b0, z_b1, src_a, dst_a, src_b, dst_b,
                  u_a0, u_a1, u_b0, u_b1,
                  src_v, dst_v, rows, acc, gsem, ssem):
    cid = lax.axis_index("c")
    tile = lax.axis_index("s")
    scr = (src_v, dst_v, rows, acc, gsem, ssem)

    def run(z0, z1, src, dst, u0, u1):
        _prop_core(z0, src, dst, u0, *scr, tile)
        _prop_core(z1, src, dst, u1, *scr, tile)

    pl.when(cid == 0)(lambda: run(z_a0, z_a1, src_a, dst_a, u_a0, u_a1))
    pl.when(cid == 1)(lambda: run(z_b0, z_b1, src_b, dst_b, u_b0, u_b1))


R = 1000
GRID = N // R


def _enc_body(gene, image, fc1w, fc1b, fc2w, fc2b, g11w, g12w, degs, degg,
              z1, z2):
    dinv_s = lax.rsqrt(degs[...])
    dinv_g = lax.rsqrt(degg[...])
    h1 = jnp.maximum(
        jnp.dot(gene[...], fc1w[...], preferred_element_type=jnp.float32)
        + fc1b[...], 0.0)
    h2 = jnp.maximum(
        jnp.dot(image[...], fc2w[...], preferred_element_type=jnp.float32)
        + fc2b[...], 0.0)
    z1[...] = jnp.dot(h1, g11w[...], preferred_element_type=jnp.float32) * dinv_s
    z2[...] = jnp.dot(h2, g12w[...], preferred_element_type=jnp.float32) * dinv_g


def _mid_body(u1, u2, degs, degg, g11b, g12b, g21w, g22w, w1,
              y3a, y3b, y4a, y4b):
    dinv_s = lax.rsqrt(degs[...])
    dinv_g = lax.rsqrt(degg[...])
    x1 = jnp.maximum(u1[...] * dinv_s + g11b[...], 0.0)
    x2 = jnp.maximum(u2[...] * dinv_g + g12b[...], 0.0)
    a = w1[0, 0]
    x = a * x1 + (1.0 - a) * x2
    y3 = jnp.dot(x, g21w[...], preferred_element_type=jnp.float32) * dinv_s
    y4 = jnp.dot(x, g22w[...], preferred_element_type=jnp.float32) * dinv_g
    y3a[...] = y3[:, :H2]
    y3b[...] = y3[:, H2:]
    y4a[...] = y4[:, :H2]
    y4b[...] = y4[:, H2:]


def _out_body(u3a, u3b, u4a, u4b, degs, degg, g21b, g22b, w2, out):
    dinv_s = lax.rsqrt(degs[...])
    dinv_g = lax.rsqrt(degg[...])
    u3 = jnp.concatenate([u3a[...], u3b[...]], axis=1)
    u4 = jnp.concatenate([u4a[...], u4b[...]], axis=1)
    a = w2[0, 0]
    out[...] = (a * (u3 * dinv_s + g21b[...])
                + (1.0 - a) * (u4 * dinv_g + g22b[...]))


def _row_spec(cols):
    return pl.BlockSpec((R, cols), lambda i: (i, 0))


def _full_spec(r, c):
    return pl.BlockSpec((r, c), lambda i: (0, 0))


_enc_call = pl.pallas_call(
    _enc_body,
    grid=(GRID,),
    in_specs=[
        _row_spec(512), _row_spec(128),
        _full_spec(512, H), _full_spec(1, H),
        _full_spec(128, H), _full_spec(1, H),
        _full_spec(H, H2), _full_spec(H, H2),
        _row_spec(1), _row_spec(1),
    ],
    out_specs=[_row_spec(H2), _row_spec(H2)],
    out_shape=[
        jax.ShapeDtypeStruct((N, H2), jnp.float32),
        jax.ShapeDtypeStruct((N, H2), jnp.float32),
    ],
)

_mid_call = pl.pallas_call(
    _mid_body,
    grid=(GRID,),
    in_specs=[
        _row_spec(H2), _row_spec(H2), _row_spec(1), _row_spec(1),
        _full_spec(1, H2), _full_spec(1, H2),
        _full_spec(H2, H), _full_spec(H2, H),
        _full_spec(1, 1),
    ],
    out_specs=[_row_spec(H2)] * 4,
    out_shape=[jax.ShapeDtypeStruct((N, H2), jnp.float32)] * 4,
)

_out_call = pl.pallas_call(
    _out_body,
    grid=(GRID,),
    in_specs=[
        _row_spec(H2), _row_spec(H2), _row_spec(H2), _row_spec(H2),
        _row_spec(1), _row_spec(1),
        _full_spec(1, OUT), _full_spec(1, OUT),
        _full_spec(1, 1),
    ],
    out_specs=_row_spec(OUT),
    out_shape=jax.ShapeDtypeStruct((N, OUT), jnp.float32),
)


def _pad_edges(edge_index):
    pad_src = jnp.zeros((E_PAD - E,), jnp.int32)
    pad_dst = jnp.full((E_PAD - E,), N, jnp.int32)
    src = jnp.concatenate([edge_index[0], pad_src]).reshape(CROWS, CHUNK)
    dst = jnp.concatenate([edge_index[1], pad_dst]).reshape(CROWS, CHUNK)
    return src, dst


def kernel(gene_data, image_data, gene_edge_index, spatial_edge_index,
           fc1_W, fc1_b, fc2_W, fc2_b,
           g11_W, g11_b, g12_W, g12_b,
           g21_W, g21_b, g22_W, g22_b, w1, w2):
    src_s, dst_s = _pad_edges(spatial_edge_index)
    src_g, dst_g = _pad_edges(gene_edge_index)
    ones_init = jnp.ones((NP, 8), jnp.float32)
    ones_tbl = jnp.ones((CHUNK, 8), jnp.float32)

    deg_s8, deg_g8 = _deg_kernel(dst_s, dst_g, ones_init, ones_tbl)
    deg_s = deg_s8[:, :1]
    deg_g = deg_g8[:, :1]

    z1, z2 = _enc_call(gene_data, image_data,
                       fc1_W, fc1_b.reshape(1, H), fc2_W, fc2_b.reshape(1, H),
                       g11_W, g12_W, deg_s, deg_g)
    u1, u2 = _prop_kernel(z1, z2, src_s, dst_s, src_g, dst_g)

    y3a, y3b, y4a, y4b = _mid_call(
        u1, u2, deg_s, deg_g,
        g11_b.reshape(1, H2), g12_b.reshape(1, H2),
        g21_W, g22_W, jnp.reshape(w1, (1, 1)))

    u3a, u3b, u4a, u4b = _prop2_kernel(y3a, y3b, y4a, y4b,
                                       src_s, dst_s, src_g, dst_g)

    return _out_call(u3a, u3b, u4a, u4b, deg_s, deg_g,
                     g21_b.reshape(1, OUT), g22_b.reshape(1, OUT),
                     jnp.reshape(w2, (1, 1)))
